# Initial kernel scaffold; baseline (speedup 1.0000x reference)
import jax
import jax.numpy as jnp
from jax.experimental import pallas as pl


def kernel(x_s, edge_index_s, batch, edge_attr_s, pos_s, n1_w1, n1_w2, n1_b2, conv1_bias, pool1_w, n2_w1, n2_w2, n2_b2, conv2_bias, pool2_w, fc1_w, fc1_b, bn1_g, bn1_b, fc2_w, fc2_b, bn2_g, bn2_b, fc3_w, fc3_b):
    raise NotImplementedError("write your pallas kernel here")


# speedup vs baseline: 24.3438x; 24.3438x over previous
import functools
import numpy as np
import jax
import jax.numpy as jnp
from jax import lax
from jax.experimental import pallas as pl
from jax.experimental.pallas import tpu as pltpu
from jax.experimental.pallas import tpu_sc as plsc

R = 200
B = 50
N = R * B
E = 320000
INDIM = 128
DIM1 = 32
DIM2 = 32
DIM3 = 512
K = 8
NCLASS = 2
KEEP1 = 100
KEEP2 = 50
EPG = E // B
NSC = 2
NSUB = 16
GPC = B // NSC
EPS_SUB = EPG // NSUB
EB = 80
NB = EPS_SUB // EB
CPB = EB // 16

_F32 = jnp.float32
_HI = lax.Precision.HIGHEST


_BLK1 = 400


def _k1_body(x_ref, pos_ref, hw_ref, w2_ref, xt_ref, h2_ref):
    hcat = jnp.maximum(
        jnp.dot(pos_ref[...], hw_ref[...], preferred_element_type=_F32,
                precision=_HI), 0.0)
    h2_ref[...] = hcat[:, K:]
    y = jnp.dot(x_ref[...], w2_ref[...], preferred_element_type=_F32,
                precision=_HI)
    xt = y[:, K * DIM1:(K + 1) * DIM1]
    for k in range(K):
        xt = xt + hcat[:, k:k + 1] * y[:, k * DIM1:(k + 1) * DIM1]
    xt_ref[...] = xt


def _k1(x_s, pos_s, hw, w2cat1):
    return pl.pallas_call(
        _k1_body,
        grid=(N // _BLK1,),
        in_specs=[
            pl.BlockSpec((_BLK1, INDIM), lambda i: (i, 0)),
            pl.BlockSpec((_BLK1, R), lambda i: (i, 0)),
            pl.BlockSpec((R, 2 * K), lambda i: (0, 0)),
            pl.BlockSpec((INDIM, (K + 1) * DIM1), lambda i: (0, 0)),
        ],
        out_specs=[
            pl.BlockSpec((_BLK1, DIM1), lambda i: (i, 0)),
            pl.BlockSpec((_BLK1, K), lambda i: (i, 0)),
        ],
        out_shape=[
            jax.ShapeDtypeStruct((N, DIM1), _F32),
            jax.ShapeDtypeStruct((N, K), _F32),
        ],
    )(x_s, pos_s, hw, w2cat1)


def _sc_wt_call(src, dst, ew):
    mesh = plsc.VectorSubcoreMesh(core_axis_name="c", subcore_axis_name="s")

    @functools.partial(
        pl.kernel, mesh=mesh,
        out_type=[
            jax.ShapeDtypeStruct((B * R * R,), _F32),
            jax.ShapeDtypeStruct((N,), _F32),
        ],
        scratch_types=[
            pltpu.VMEM_SHARED((GPC * R * R,), _F32),
            pltpu.VMEM_SHARED((GPC * R,), _F32),
            pltpu.VMEM((R * R,), _F32),
            pltpu.VMEM((EPS_SUB,), jnp.int32),
            pltpu.VMEM((EPS_SUB,), jnp.int32),
            pltpu.VMEM((EPS_SUB,), _F32),
            pltpu.VMEM((EB,), jnp.int32),
            pltpu.VMEM((EB,), jnp.int32),
            pltpu.VMEM((EB,), _F32),
            pltpu.VMEM((EB,), _F32),
        ],
    )
    def sck1(src_h, dst_h, ew_h, wt_out, cnt_out, wt_sp, cnt_sp, buf_v,
             src_v, dst_v, ew_v, widx_v, cidx_v, wval_v, ones_v):
        c = lax.axis_index("c")
        s = lax.axis_index("s")

        def zb(i, carry):
            buf_v[pl.ds(i * 16, 16)] = jnp.zeros((16,), _F32)
            return carry
        lax.fori_loop(0, R * R // 16, zb, 0)
        for gl in range(GPC):
            @pl.when(gl % NSUB == s)
            def _():
                pltpu.sync_copy(buf_v, wt_sp.at[pl.ds(gl * R * R, R * R)])
        @pl.when(s == 0)
        def _():
            pltpu.sync_copy(buf_v.at[pl.ds(0, GPC * R)], cnt_sp)
        for t in range(CPB):
            ones_v[pl.ds(t * 16, 16)] = jnp.ones((16,), _F32)
        plsc.subcore_barrier()

        def body(gl, carry):
            g_abs = c * GPC + gl
            ebase = g_abs * EPG + s * EPS_SUB
            pltpu.sync_copy(src_h.at[pl.ds(ebase, EPS_SUB)], src_v)
            pltpu.sync_copy(dst_h.at[pl.ds(ebase, EPS_SUB)], dst_v)
            pltpu.sync_copy(ew_h.at[pl.ds(ebase, EPS_SUB)], ew_v)
            nb = g_abs * R
            cb = c * (GPC * R)
            for b in range(NB):
                for t in range(CPB):
                    j = b * CPB + t
                    sv = src_v[pl.ds(j * 16, 16)]
                    dv = dst_v[pl.ds(j * 16, 16)]
                    sl = sv - nb
                    dl = dv - nb
                    widx_v[pl.ds(t * 16, 16)] = gl * (R * R) + dl * R + sl
                    cidx_v[pl.ds(t * 16, 16)] = dv - cb
                    wval_v[pl.ds(t * 16, 16)] = ew_v[pl.ds(j * 16, 16)]
                pltpu.sync_copy(wval_v, wt_sp.at[widx_v], add=True)
                pltpu.sync_copy(ones_v, cnt_sp.at[cidx_v], add=True)
            return carry

        lax.fori_loop(0, GPC, body, 0)
        plsc.subcore_barrier()
        for gl in range(GPC):
            @pl.when(gl % NSUB == s)
            def _():
                g_abs = c * GPC + gl
                pltpu.sync_copy(wt_sp.at[pl.ds(gl * R * R, R * R)], buf_v)
                pltpu.sync_copy(buf_v,
                                wt_out.at[pl.ds(g_abs * R * R, R * R)])
        @pl.when(s == 0)
        def _():
            pltpu.sync_copy(cnt_sp, buf_v.at[pl.ds(0, GPC * R)])
            pltpu.sync_copy(buf_v.at[pl.ds(0, GPC * R)],
                            cnt_out.at[pl.ds(c * GPC * R, GPC * R)])

    return sck1(src, dst, ew)


def _k2_body(wt_ref, xt_ref, cnt_ref, h2_ref, w_ref, b_ref,
             xp_ref, h2s_ref, x1s_ref, sig1_ref, at_ref):
    agg = jnp.dot(wt_ref[0], xt_ref[...], preferred_element_type=_F32,
                  precision=_HI)
    cnt = cnt_ref[0]
    x1 = jnp.where(cnt > 0, agg / jnp.maximum(cnt, 1.0), 0.0) + b_ref[...]
    w = w_ref[...]
    nrm = jnp.sqrt(jnp.sum(w * w))
    zrow = lax.dot_general(w, x1, (((1,), (1,)), ((), ())),
                           preferred_element_type=_F32, precision=_HI) / nrm
    one_r = jnp.ones((1, R), _F32)
    si = lax.dot_general(zrow, one_r, (((0,), (0,)), ((), ())),
                         preferred_element_type=_F32, precision=_HI)
    sj = lax.dot_general(one_r, zrow, (((0,), (0,)), ((), ())),
                         preferred_element_type=_F32, precision=_HI)
    si = jax.nn.sigmoid(si)
    sj = jax.nn.sigmoid(sj)
    i0 = lax.broadcasted_iota(jnp.int32, (R, R), 0)
    i1 = lax.broadcasted_iota(jnp.int32, (R, R), 1)
    beats = ((sj > si) | ((sj == si) & (i1 < i0))).astype(_F32)
    rank = jnp.sum(beats, axis=1, keepdims=True)
    rmat = lax.dot_general(jnp.ones((KEEP1, 1), _F32), rank,
                           (((1,), (1,)), ((), ())),
                           preferred_element_type=_F32, precision=_HI)
    piota = lax.broadcasted_iota(jnp.int32, (KEEP1, R), 0).astype(_F32)
    perm = (rmat == piota).astype(_F32)
    scol = si[:, 0:1]
    xp = jnp.dot(perm, x1 * scol, preferred_element_type=_F32, precision=_HI)
    xp_ref[0] = xp
    h2s_ref[0] = jnp.dot(perm, h2_ref[...], preferred_element_type=_F32,
                         precision=_HI)
    sc1 = jnp.dot(perm, scol, preferred_element_type=_F32, precision=_HI)
    sig1_ref[0] = jax.nn.sigmoid(sc1)
    mx = jnp.max(xp, axis=0, keepdims=True)
    mn = jnp.sum(xp, axis=0, keepdims=True) / _F32(KEEP1)
    x1s_ref[0] = jnp.concatenate([mx, mn], axis=1)
    pwt = jnp.dot(perm, wt_ref[0], preferred_element_type=_F32, precision=_HI)
    at_ref[0] = lax.dot_general(pwt, perm, (((1,), (1,)), ((), ())),
                                preferred_element_type=_F32, precision=_HI)


def _k2(WT, xt1, cnt3, h2full, pw, cb):
    return pl.pallas_call(
        _k2_body,
        grid=(B,),
        in_specs=[
            pl.BlockSpec((1, R, R), lambda g: (g, 0, 0)),
            pl.BlockSpec((R, DIM1), lambda g: (g, 0)),
            pl.BlockSpec((1, R, 1), lambda g: (g, 0, 0)),
            pl.BlockSpec((R, K), lambda g: (g, 0)),
            pl.BlockSpec((1, DIM1), lambda g: (0, 0)),
            pl.BlockSpec((1, DIM1), lambda g: (0, 0)),
        ],
        out_specs=[
            pl.BlockSpec((1, KEEP1, DIM1), lambda g: (g, 0, 0)),
            pl.BlockSpec((1, KEEP1, K), lambda g: (g, 0, 0)),
            pl.BlockSpec((1, 1, 2 * DIM1), lambda g: (g, 0, 0)),
            pl.BlockSpec((1, KEEP1, 1), lambda g: (g, 0, 0)),
            pl.BlockSpec((1, KEEP1, KEEP1), lambda g: (g, 0, 0)),
        ],
        out_shape=[
            jax.ShapeDtypeStruct((B, KEEP1, DIM1), _F32),
            jax.ShapeDtypeStruct((B, KEEP1, K), _F32),
            jax.ShapeDtypeStruct((B, 1, 2 * DIM1), _F32),
            jax.ShapeDtypeStruct((B, KEEP1, 1), _F32),
            jax.ShapeDtypeStruct((B, KEEP1, KEEP1), _F32),
        ],
    )(WT, xt1, cnt3, h2full, pw, cb)


def _k3_body(at_ref, xp1_ref, h2s_ref, x1s_ref, w2_ref, cb_ref, pw_ref,
             f1w_ref, f1b_ref, g1_ref, b1_ref, f2w_ref, f2b_ref, g2_ref,
             b2_ref, f3w_ref, f3b_ref, logp_ref, sig2_ref):
    i0 = lax.broadcasted_iota(jnp.int32, (KEEP1, KEEP1), 0)
    i1 = lax.broadcasted_iota(jnp.int32, (KEEP1, KEEP1), 1)
    eye = (i0 == i1).astype(_F32)
    aaug = at_ref[0] + eye
    a2 = jnp.dot(aaug, aaug, preferred_element_type=_F32, precision=_HI)
    a2 = a2 * (1.0 - eye)
    cnt2 = jnp.sum((a2 != 0.0).astype(_F32), axis=1, keepdims=True)
    xp = xp1_ref[0]
    y2 = jnp.dot(xp, w2_ref[...], preferred_element_type=_F32, precision=_HI)
    h2s = h2s_ref[0]
    xt2 = y2[:, K * DIM2:(K + 1) * DIM2]
    for k in range(K):
        xt2 = xt2 + h2s[:, k:k + 1] * y2[:, k * DIM2:(k + 1) * DIM2]
    num = jnp.dot(a2, xt2, preferred_element_type=_F32, precision=_HI)
    x2 = jnp.where(cnt2 > 0, num / jnp.maximum(cnt2, 1.0), 0.0) + cb_ref[...]
    w = pw_ref[...]
    nrm = jnp.sqrt(jnp.sum(w * w))
    zrow = lax.dot_general(w, x2, (((1,), (1,)), ((), ())),
                           preferred_element_type=_F32, precision=_HI) / nrm
    one_k = jnp.ones((1, KEEP1), _F32)
    si = lax.dot_general(zrow, one_k, (((0,), (0,)), ((), ())),
                         preferred_element_type=_F32, precision=_HI)
    sj = lax.dot_general(one_k, zrow, (((0,), (0,)), ((), ())),
                         preferred_element_type=_F32, precision=_HI)
    si = jax.nn.sigmoid(si)
    sj = jax.nn.sigmoid(sj)
    j0 = lax.broadcasted_iota(jnp.int32, (KEEP1, KEEP1), 0)
    j1 = lax.broadcasted_iota(jnp.int32, (KEEP1, KEEP1), 1)
    beats = ((sj > si) | ((sj == si) & (j1 < j0))).astype(_F32)
    rank = jnp.sum(beats, axis=1, keepdims=True)
    rmat = lax.dot_general(jnp.ones((KEEP2, 1), _F32), rank,
                           (((1,), (1,)), ((), ())),
                           preferred_element_type=_F32, precision=_HI)
    piota = lax.broadcasted_iota(jnp.int32, (KEEP2, KEEP1), 0).astype(_F32)
    perm = (rmat == piota).astype(_F32)
    scol = si[:, 0:1]
    xp2 = jnp.dot(perm, x2 * scol, preferred_element_type=_F32, precision=_HI)
    sc2 = jnp.dot(perm, scol, preferred_element_type=_F32, precision=_HI)
    sig2_ref[0] = jax.nn.sigmoid(sc2)
    mx = jnp.max(xp2, axis=0, keepdims=True)
    mn = jnp.sum(xp2, axis=0, keepdims=True) / _F32(KEEP2)
    xg = jnp.concatenate([x1s_ref[0], mx, mn], axis=1)
    inv_bn = 1.0 / jnp.sqrt(_F32(1.0 + 1e-5))
    t = jnp.maximum(jnp.dot(xg, f1w_ref[...], preferred_element_type=_F32,
                            precision=_HI) + f1b_ref[...], 0.0)
    t = (t * inv_bn) * g1_ref[...] + b1_ref[...]
    t = jnp.maximum(jnp.dot(t, f2w_ref[...], preferred_element_type=_F32,
                            precision=_HI) + f2b_ref[...], 0.0)
    t = (t * inv_bn) * g2_ref[...] + b2_ref[...]
    logits = jnp.dot(t, f3w_ref[...], preferred_element_type=_F32,
                     precision=_HI) + f3b_ref[...]
    m = jnp.max(logits, axis=1, keepdims=True)
    lse = m + jnp.log(jnp.sum(jnp.exp(logits - m), axis=1, keepdims=True))
    logp_ref[0] = logits - lse


def _k3(At, xp1, h2sel, x1s, w2cat2, cb, pw, f1w, f1b, g1, b1, f2w, f2b,
        g2, b2, f3w, f3b):
    full = lambda a, b: pl.BlockSpec((a, b), lambda g: (0, 0))
    return pl.pallas_call(
        _k3_body,
        grid=(B,),
        in_specs=[
            pl.BlockSpec((1, KEEP1, KEEP1), lambda g: (g, 0, 0)),
            pl.BlockSpec((1, KEEP1, DIM1), lambda g: (g, 0, 0)),
            pl.BlockSpec((1, KEEP1, K), lambda g: (g, 0, 0)),
            pl.BlockSpec((1, 1, 2 * DIM1), lambda g: (g, 0, 0)),
            full(DIM1, (K + 1) * DIM2),
            full(1, DIM2),
            full(1, DIM2),
            full(2 * (DIM1 + DIM2), DIM2),
            full(1, DIM2),
            full(1, DIM2),
            full(1, DIM2),
            full(DIM2, DIM3),
            full(1, DIM3),
            full(1, DIM3),
            full(1, DIM3),
            full(DIM3, NCLASS),
            full(1, NCLASS),
        ],
        out_specs=[
            pl.BlockSpec((1, 1, NCLASS), lambda g: (g, 0, 0)),
            pl.BlockSpec((1, KEEP2, 1), lambda g: (g, 0, 0)),
        ],
        out_shape=[
            jax.ShapeDtypeStruct((B, 1, NCLASS), _F32),
            jax.ShapeDtypeStruct((B, KEEP2, 1), _F32),
        ],
    )(At, xp1, h2sel, x1s, w2cat2, cb, pw, f1w, f1b, g1, b1, f2w, f2b,
      g2, b2, f3w, f3b)


def kernel(x_s, edge_index_s, batch, edge_attr_s, pos_s, n1_w1, n1_w2, n1_b2,
           conv1_bias, pool1_w, n2_w1, n2_w2, n2_b2, conv2_bias, pool2_w,
           fc1_w, fc1_b, bn1_g, bn1_b, fc2_w, fc2_b, bn2_g, bn2_b,
           fc3_w, fc3_b):
    src = edge_index_s[0]
    dst = edge_index_s[1]
    ew = edge_attr_s.reshape(E)
    hw = jnp.concatenate([n1_w1, n2_w1], axis=1)
    w2cat1 = jnp.concatenate([n1_w2, n1_b2[None, :]], axis=0) \
        .reshape(K + 1, INDIM, DIM1).transpose(1, 0, 2) \
        .reshape(INDIM, (K + 1) * DIM1)
    w2cat2 = jnp.concatenate([n2_w2, n2_b2[None, :]], axis=0) \
        .reshape(K + 1, DIM1, DIM2).transpose(1, 0, 2) \
        .reshape(DIM1, (K + 1) * DIM2)
    xt1, h2full = _k1(x_s, pos_s, hw, w2cat1)
    wt_flat, cnt1 = _sc_wt_call(src, dst, ew)
    WT = wt_flat.reshape(B, R, R)
    cnt3 = cnt1.reshape(B, R, 1)
    xp1, h2sel, x1s, sig1, At = _k2(
        WT, xt1, cnt3, h2full, pool1_w.reshape(1, DIM1),
        conv1_bias.reshape(1, DIM1))
    logp3, sig2 = _k3(
        At, xp1, h2sel, x1s, w2cat2, conv2_bias.reshape(1, DIM2),
        pool2_w.reshape(1, DIM2), fc1_w, fc1_b.reshape(1, DIM2),
        bn1_g.reshape(1, DIM2), bn1_b.reshape(1, DIM2), fc2_w,
        fc2_b.reshape(1, DIM3), bn2_g.reshape(1, DIM3),
        bn2_b.reshape(1, DIM3), fc3_w, fc3_b.reshape(1, NCLASS))
    return (logp3.reshape(B, NCLASS), pool1_w, pool2_w,
            sig1.reshape(B, KEEP1), sig2.reshape(B, KEEP2))


# speedup vs baseline: 25.4637x; 1.0460x over previous
import functools
import numpy as np
import jax
import jax.numpy as jnp
from jax import lax
from jax.experimental import pallas as pl
from jax.experimental.pallas import tpu as pltpu
from jax.experimental.pallas import tpu_sc as plsc

R = 200
B = 50
N = R * B
E = 320000
INDIM = 128
DIM1 = 32
DIM2 = 32
DIM3 = 512
K = 8
NCLASS = 2
KEEP1 = 100
KEEP2 = 50
EPG = E // B
NSC = 2
NSUB = 16
GPC = B // NSC
EPS_SUB = EPG // NSUB
EB = 80
NB = EPS_SUB // EB
CPB = EB // 16

_F32 = jnp.float32
_HI = lax.Precision.HIGHEST


_BLK1 = 400


def _k1_body(x_ref, pos_ref, hw_ref, w2_ref, xt_ref, h2_ref):
    hcat = jnp.maximum(
        jnp.dot(pos_ref[...], hw_ref[...], preferred_element_type=_F32,
                precision=_HI), 0.0)
    h2_ref[...] = hcat[:, K:]
    y = jnp.dot(x_ref[...], w2_ref[...], preferred_element_type=_F32,
                precision=_HI)
    xt = y[:, K * DIM1:(K + 1) * DIM1]
    for k in range(K):
        xt = xt + hcat[:, k:k + 1] * y[:, k * DIM1:(k + 1) * DIM1]
    xt_ref[...] = xt


def _k1(x_s, pos_s, hw, w2cat1):
    return pl.pallas_call(
        _k1_body,
        grid=(N // _BLK1,),
        in_specs=[
            pl.BlockSpec((_BLK1, INDIM), lambda i: (i, 0)),
            pl.BlockSpec((_BLK1, R), lambda i: (i, 0)),
            pl.BlockSpec((R, 2 * K), lambda i: (0, 0)),
            pl.BlockSpec((INDIM, (K + 1) * DIM1), lambda i: (0, 0)),
        ],
        out_specs=[
            pl.BlockSpec((_BLK1, DIM1), lambda i: (i, 0)),
            pl.BlockSpec((_BLK1, K), lambda i: (i, 0)),
        ],
        out_shape=[
            jax.ShapeDtypeStruct((N, DIM1), _F32),
            jax.ShapeDtypeStruct((N, K), _F32),
        ],
    )(x_s, pos_s, hw, w2cat1)


def _sc_wt_call(src, dst, ew):
    mesh = plsc.VectorSubcoreMesh(core_axis_name="c", subcore_axis_name="s")

    @functools.partial(
        pl.kernel, mesh=mesh,
        out_type=[
            jax.ShapeDtypeStruct((B * R * R,), _F32),
            jax.ShapeDtypeStruct((N,), _F32),
        ],
        scratch_types=[
            pltpu.VMEM_SHARED((GPC * R * R,), _F32),
            pltpu.VMEM_SHARED((GPC * R,), _F32),
            pltpu.VMEM((R * R,), _F32),
            pltpu.VMEM((EPS_SUB,), jnp.int32),
            pltpu.VMEM((EPS_SUB,), jnp.int32),
            pltpu.VMEM((EPS_SUB,), _F32),
            pltpu.VMEM((EB,), jnp.int32),
            pltpu.VMEM((EB,), jnp.int32),
            pltpu.VMEM((EB,), _F32),
            pltpu.VMEM((EB,), _F32),
        ],
    )
    def sck1(src_h, dst_h, ew_h, wt_out, cnt_out, wt_sp, cnt_sp, buf_v,
             src_v, dst_v, ew_v, widx_v, cidx_v, wval_v, ones_v):
        c = lax.axis_index("c")
        s = lax.axis_index("s")

        def zb(i, carry):
            buf_v[pl.ds(i * 16, 16)] = jnp.zeros((16,), _F32)
            return carry
        lax.fori_loop(0, R * R // 16, zb, 0)
        for gl in range(GPC):
            @pl.when(gl % NSUB == s)
            def _():
                pltpu.sync_copy(buf_v, wt_sp.at[pl.ds(gl * R * R, R * R)])
        @pl.when(s == 0)
        def _():
            pltpu.sync_copy(buf_v.at[pl.ds(0, GPC * R)], cnt_sp)
        for t in range(CPB):
            ones_v[pl.ds(t * 16, 16)] = jnp.ones((16,), _F32)
        plsc.subcore_barrier()

        def body(gl, carry):
            g_abs = c * GPC + gl
            ebase = g_abs * EPG + s * EPS_SUB
            pltpu.sync_copy(src_h.at[pl.ds(ebase, EPS_SUB)], src_v)
            pltpu.sync_copy(dst_h.at[pl.ds(ebase, EPS_SUB)], dst_v)
            pltpu.sync_copy(ew_h.at[pl.ds(ebase, EPS_SUB)], ew_v)
            nb = g_abs * R
            cb = c * (GPC * R)
            for b in range(NB):
                for t in range(CPB):
                    j = b * CPB + t
                    sv = src_v[pl.ds(j * 16, 16)]
                    dv = dst_v[pl.ds(j * 16, 16)]
                    sl = sv - nb
                    dl = dv - nb
                    widx_v[pl.ds(t * 16, 16)] = gl * (R * R) + dl * R + sl
                    cidx_v[pl.ds(t * 16, 16)] = dv - cb
                    wval_v[pl.ds(t * 16, 16)] = ew_v[pl.ds(j * 16, 16)]
                pltpu.sync_copy(wval_v, wt_sp.at[widx_v], add=True)
                pltpu.sync_copy(ones_v, cnt_sp.at[cidx_v], add=True)
            return carry

        lax.fori_loop(0, GPC, body, 0)
        plsc.subcore_barrier()
        for gl in range(GPC):
            @pl.when(gl % NSUB == s)
            def _():
                g_abs = c * GPC + gl
                pltpu.sync_copy(wt_sp.at[pl.ds(gl * R * R, R * R)], buf_v)
                pltpu.sync_copy(buf_v,
                                wt_out.at[pl.ds(g_abs * R * R, R * R)])
        @pl.when(s == 0)
        def _():
            pltpu.sync_copy(cnt_sp, buf_v.at[pl.ds(0, GPC * R)])
            pltpu.sync_copy(buf_v.at[pl.ds(0, GPC * R)],
                            cnt_out.at[pl.ds(c * GPC * R, GPC * R)])

    return sck1(src, dst, ew)


G23 = 5


def _rank_and_perm(x, w_row, nrm, n, keep):
    zrow = lax.dot_general(w_row, x, (((1,), (1,)), ((), ())),
                           preferred_element_type=_F32, precision=_HI) / nrm
    one_n = jnp.ones((1, n), _F32)
    si = lax.dot_general(zrow, one_n, (((0,), (0,)), ((), ())),
                         preferred_element_type=_F32, precision=_HI)
    sj = lax.dot_general(one_n, zrow, (((0,), (0,)), ((), ())),
                         preferred_element_type=_F32, precision=_HI)
    si = jax.nn.sigmoid(si)
    sj = jax.nn.sigmoid(sj)
    i0 = lax.broadcasted_iota(jnp.int32, (n, n), 0)
    i1 = lax.broadcasted_iota(jnp.int32, (n, n), 1)
    beats = ((sj > si) | ((sj == si) & (i1 < i0))).astype(_F32)
    rank = jnp.sum(beats, axis=1, keepdims=True)
    rmat = lax.dot_general(jnp.ones((keep, 1), _F32), rank,
                           (((1,), (1,)), ((), ())),
                           preferred_element_type=_F32, precision=_HI)
    piota = lax.broadcasted_iota(jnp.int32, (keep, n), 0).astype(_F32)
    perm = (rmat == piota).astype(_F32)
    return si[:, 0:1], perm


def _k23_body(wt_ref, xt_ref, cnt_ref, h2_ref, pw1_ref, cb1_ref, w2_ref,
              cb2_ref, pw2_ref, f1w_ref, f1b_ref, g1_ref, b1_ref, f2w_ref,
              f2b_ref, g2_ref, b2_ref, f3w_ref, f3b_ref,
              logp_ref, sig1_ref, sig2_ref):
    pw1 = pw1_ref[...]
    nrm1 = jnp.sqrt(jnp.sum(pw1 * pw1))
    pw2 = pw2_ref[...]
    nrm2 = jnp.sqrt(jnp.sum(pw2 * pw2))
    ey0 = lax.broadcasted_iota(jnp.int32, (KEEP1, KEEP1), 0)
    ey1 = lax.broadcasted_iota(jnp.int32, (KEEP1, KEEP1), 1)
    eye = (ey0 == ey1).astype(_F32)
    inv_bn = 1.0 / jnp.sqrt(_F32(1.0 + 1e-5))
    for i in range(G23):
        wt = wt_ref[i]
        agg = jnp.dot(wt, xt_ref[pl.ds(i * R, R), :],
                      preferred_element_type=_F32, precision=_HI)
        cnt = cnt_ref[i]
        x1 = jnp.where(cnt > 0, agg / jnp.maximum(cnt, 1.0), 0.0) \
            + cb1_ref[...]
        scol, perm = _rank_and_perm(x1, pw1, nrm1, R, KEEP1)
        xp = jnp.dot(perm, x1 * scol, preferred_element_type=_F32,
                     precision=_HI)
        h2s = jnp.dot(perm, h2_ref[pl.ds(i * R, R), :],
                      preferred_element_type=_F32, precision=_HI)
        sc1 = jnp.dot(perm, scol, preferred_element_type=_F32, precision=_HI)
        sig1_ref[i] = jax.nn.sigmoid(sc1)
        mx1 = jnp.max(xp, axis=0, keepdims=True)
        mn1 = jnp.sum(xp, axis=0, keepdims=True) / _F32(KEEP1)
        pwt = jnp.dot(perm, wt, preferred_element_type=_F32, precision=_HI)
        at = lax.dot_general(pwt, perm, (((1,), (1,)), ((), ())),
                             preferred_element_type=_F32, precision=_HI)
        aaug = at + eye
        a2 = jnp.dot(aaug, aaug, preferred_element_type=_F32, precision=_HI)
        a2 = a2 * (1.0 - eye)
        cnt2 = jnp.sum((a2 != 0.0).astype(_F32), axis=1, keepdims=True)
        y2 = jnp.dot(xp, w2_ref[...], preferred_element_type=_F32,
                     precision=_HI)
        xt2 = y2[:, K * DIM2:(K + 1) * DIM2]
        for k in range(K):
            xt2 = xt2 + h2s[:, k:k + 1] * y2[:, k * DIM2:(k + 1) * DIM2]
        num = jnp.dot(a2, xt2, preferred_element_type=_F32, precision=_HI)
        x2 = jnp.where(cnt2 > 0, num / jnp.maximum(cnt2, 1.0), 0.0) \
            + cb2_ref[...]
        scol2, perm2 = _rank_and_perm(x2, pw2, nrm2, KEEP1, KEEP2)
        xp2 = jnp.dot(perm2, x2 * scol2, preferred_element_type=_F32,
                      precision=_HI)
        sc2 = jnp.dot(perm2, scol2, preferred_element_type=_F32,
                      precision=_HI)
        sig2_ref[i] = jax.nn.sigmoid(sc2)
        mx2 = jnp.max(xp2, axis=0, keepdims=True)
        mn2 = jnp.sum(xp2, axis=0, keepdims=True) / _F32(KEEP2)
        xg = jnp.concatenate([mx1, mn1, mx2, mn2], axis=1)
        t = jnp.maximum(jnp.dot(xg, f1w_ref[...], preferred_element_type=_F32,
                                precision=_HI) + f1b_ref[...], 0.0)
        t = (t * inv_bn) * g1_ref[...] + b1_ref[...]
        t = jnp.maximum(jnp.dot(t, f2w_ref[...], preferred_element_type=_F32,
                                precision=_HI) + f2b_ref[...], 0.0)
        t = (t * inv_bn) * g2_ref[...] + b2_ref[...]
        logits = jnp.dot(t, f3w_ref[...], preferred_element_type=_F32,
                         precision=_HI) + f3b_ref[...]
        m = jnp.max(logits, axis=1, keepdims=True)
        lse = m + jnp.log(jnp.sum(jnp.exp(logits - m), axis=1, keepdims=True))
        logp_ref[i] = logits - lse


def _k23(WT, xt1, cnt3, h2full, pw1, cb1, w2cat2, cb2, pw2, f1w, f1b, g1, b1,
         f2w, f2b, g2, b2, f3w, f3b):
    full = lambda a, b: pl.BlockSpec((a, b), lambda g: (0, 0))
    return pl.pallas_call(
        _k23_body,
        grid=(B // G23,),
        in_specs=[
            pl.BlockSpec((G23, R, R), lambda g: (g, 0, 0)),
            pl.BlockSpec((G23 * R, DIM1), lambda g: (g, 0)),
            pl.BlockSpec((G23, R, 1), lambda g: (g, 0, 0)),
            pl.BlockSpec((G23 * R, K), lambda g: (g, 0)),
            full(1, DIM1),
            full(1, DIM1),
            full(DIM1, (K + 1) * DIM2),
            full(1, DIM2),
            full(1, DIM2),
            full(2 * (DIM1 + DIM2), DIM2),
            full(1, DIM2),
            full(1, DIM2),
            full(1, DIM2),
            full(DIM2, DIM3),
            full(1, DIM3),
            full(1, DIM3),
            full(1, DIM3),
            full(DIM3, NCLASS),
            full(1, NCLASS),
        ],
        out_specs=[
            pl.BlockSpec((G23, 1, NCLASS), lambda g: (g, 0, 0)),
            pl.BlockSpec((G23, KEEP1, 1), lambda g: (g, 0, 0)),
            pl.BlockSpec((G23, KEEP2, 1), lambda g: (g, 0, 0)),
        ],
        out_shape=[
            jax.ShapeDtypeStruct((B, 1, NCLASS), _F32),
            jax.ShapeDtypeStruct((B, KEEP1, 1), _F32),
            jax.ShapeDtypeStruct((B, KEEP2, 1), _F32),
        ],
    )(WT, xt1, cnt3, h2full, pw1, cb1, w2cat2, cb2, pw2, f1w, f1b, g1, b1,
      f2w, f2b, g2, b2, f3w, f3b)


def kernel(x_s, edge_index_s, batch, edge_attr_s, pos_s, n1_w1, n1_w2, n1_b2,
           conv1_bias, pool1_w, n2_w1, n2_w2, n2_b2, conv2_bias, pool2_w,
           fc1_w, fc1_b, bn1_g, bn1_b, fc2_w, fc2_b, bn2_g, bn2_b,
           fc3_w, fc3_b):
    src = edge_index_s[0]
    dst = edge_index_s[1]
    ew = edge_attr_s.reshape(E)
    hw = jnp.concatenate([n1_w1, n2_w1], axis=1)
    w2cat1 = jnp.concatenate([n1_w2, n1_b2[None, :]], axis=0) \
        .reshape(K + 1, INDIM, DIM1).transpose(1, 0, 2) \
        .reshape(INDIM, (K + 1) * DIM1)
    w2cat2 = jnp.concatenate([n2_w2, n2_b2[None, :]], axis=0) \
        .reshape(K + 1, DIM1, DIM2).transpose(1, 0, 2) \
        .reshape(DIM1, (K + 1) * DIM2)
    xt1, h2full = _k1(x_s, pos_s, hw, w2cat1)
    wt_flat, cnt1 = _sc_wt_call(src, dst, ew)
    WT = wt_flat.reshape(B, R, R)
    cnt3 = cnt1.reshape(B, R, 1)
    logp3, sig1, sig2 = _k23(
        WT, xt1, cnt3, h2full, pool1_w.reshape(1, DIM1),
        conv1_bias.reshape(1, DIM1), w2cat2, conv2_bias.reshape(1, DIM2),
        pool2_w.reshape(1, DIM2), fc1_w, fc1_b.reshape(1, DIM2),
        bn1_g.reshape(1, DIM2), bn1_b.reshape(1, DIM2), fc2_w,
        fc2_b.reshape(1, DIM3), bn2_g.reshape(1, DIM3),
        bn2_b.reshape(1, DIM3), fc3_w, fc3_b.reshape(1, NCLASS))
    return (logp3.reshape(B, NCLASS), pool1_w, pool2_w,
            sig1.reshape(B, KEEP1), sig2.reshape(B, KEEP2))
